# Initial kernel scaffold; baseline (speedup 1.0000x reference)
import jax
import jax.numpy as jnp
from jax.experimental import pallas as pl


def kernel(x, edge_index, Ws1, Wn1, b1, Ws2, Wn2, b2, W_fc1, b_fc1, W_fc2, b_fc2):
    raise NotImplementedError("write your pallas kernel here")


# speedup vs baseline: 2.8933x; 2.8933x over previous
import functools

import jax
import jax.numpy as jnp
from jax import lax
from jax.experimental import pallas as pl
from jax.experimental.pallas import tpu as pltpu
from jax.experimental.pallas import tpu_sc as plsc

N_NODES = 10000
NP = 10240
D1P = 128
D2P = 128
CHUNK = 128
NC, NS = 2, 16
NW = NC * NS


def _sc_agg(p, src2d, dst2d, dp, ec):
    rows_per_tile = NP // NS
    n_zero = rows_per_tile // CHUNK
    mesh = plsc.VectorSubcoreMesh(core_axis_name="c", subcore_axis_name="s")

    @functools.partial(
        pl.kernel,
        out_type=jax.ShapeDtypeStruct((NC, NP, dp), jnp.float32),
        mesh=mesh,
        scratch_types=[
            pltpu.VMEM((ec, CHUNK), jnp.int32),
            pltpu.VMEM((ec, CHUNK), jnp.int32),
            pltpu.VMEM((CHUNK, dp), jnp.float32),
            pltpu.VMEM_SHARED((NP, dp), jnp.float32),
        ],
    )
    def k(p_hbm, src_hbm, dst_hbm, out_hbm, src_v, dst_v, rows_v, acc):
        c = lax.axis_index("c")
        s = lax.axis_index("s")
        wid = s * NC + c

        zero16 = jnp.zeros((16,), jnp.float32)

        def zbody(r, carry):
            for jj in range(dp // 16):
                rows_v[r, pl.ds(jj * 16, 16)] = zero16
            return carry

        lax.fori_loop(0, CHUNK, zbody, 0)
        for b in range(n_zero):
            pltpu.sync_copy(rows_v, acc.at[pl.ds(s * rows_per_tile + b * CHUNK, CHUNK)])
        plsc.subcore_barrier()

        pltpu.sync_copy(src_hbm.at[pl.ds(wid * ec, ec)], src_v)
        pltpu.sync_copy(dst_hbm.at[pl.ds(wid * ec, ec)], dst_v)

        def ebody(j, carry):
            pltpu.sync_copy(p_hbm.at[src_v.at[j]], rows_v)
            pltpu.sync_copy(rows_v, acc.at[dst_v.at[j]], add=True)
            return carry

        lax.fori_loop(0, ec, ebody, 0)
        plsc.subcore_barrier()

        pltpu.sync_copy(
            acc.at[pl.ds(s * rows_per_tile, rows_per_tile)],
            out_hbm.at[c, pl.ds(s * rows_per_tile, rows_per_tile)],
        )

    return k(p, src2d, dst2d)


def _proj1_body(x_ref, w_ref, o_ref):
    ones_col = (lax.broadcasted_iota(jnp.int32, (1, D1P), 1) == 100).astype(jnp.float32)
    o_ref[...] = jnp.dot(x_ref[...], w_ref[...],
                         preferred_element_type=jnp.float32, precision=jax.lax.Precision.HIGHEST) + ones_col


def _mid_body(x_ref, agg_ref, ws1_ref, b1_ref, wn2_ref, ws2_ref, p2_ref, s2_ref):
    a = agg_ref[0] + agg_ref[1]
    deg = jnp.clip(a[:, 100:101], 1.0, None)
    mean1 = a / deg
    h1 = jnp.maximum(
        jnp.dot(x_ref[...], ws1_ref[...], preferred_element_type=jnp.float32, precision=jax.lax.Precision.HIGHEST)
        + b1_ref[...] + mean1, 0.0)
    is_ones = lax.broadcasted_iota(jnp.int32, h1.shape, 1) == 100
    h1 = jnp.where(is_ones, 1.0, h1)
    p2_ref[...] = jnp.dot(h1, wn2_ref[...], preferred_element_type=jnp.float32, precision=jax.lax.Precision.HIGHEST)
    s2_ref[...] = jnp.dot(h1, ws2_ref[...], preferred_element_type=jnp.float32, precision=jax.lax.Precision.HIGHEST)


def _tail_body(s2_ref, agg_ref, b2_ref, wfc1_ref, bfc1_ref, wfc2_ref, bfc2_ref,
               hg_ref, o_ref, *, bs, ngrid):
    i = pl.program_id(0)
    a = agg_ref[0] + agg_ref[1]
    deg = jnp.clip(a[:, 20:21], 1.0, None)
    h2 = jnp.maximum(s2_ref[...] + a / deg + b2_ref[...], 0.0)
    row = lax.broadcasted_iota(jnp.int32, h2.shape, 0) + i * bs
    h2 = jnp.where(row < N_NODES, h2, 0.0)
    psum = jnp.sum(h2, axis=0, keepdims=True)

    @pl.when(i == 0)
    def _():
        hg_ref[...] = psum

    @pl.when(i > 0)
    def _():
        hg_ref[...] = hg_ref[...] + psum

    @pl.when(i == ngrid - 1)
    def _():
        hg = hg_ref[...] * (1.0 / N_NODES)
        t = jnp.maximum(
            jnp.dot(hg, wfc1_ref[...], preferred_element_type=jnp.float32, precision=jax.lax.Precision.HIGHEST)
            + bfc1_ref[...], 0.0)
        o_ref[...] = (jnp.dot(t, wfc2_ref[...], preferred_element_type=jnp.float32, precision=jax.lax.Precision.HIGHEST)
                      + bfc2_ref[...])


def kernel(x, edge_index, Ws1, Wn1, b1, Ws2, Wn2, b2, W_fc1, b_fc1, W_fc2, b_fc2):
    n, d_in = x.shape
    e = edge_index.shape[1]

    xp = jnp.pad(x, ((0, NP - n), (0, 0)))
    src = edge_index[0].astype(jnp.int32)
    dst = edge_index[1].astype(jnp.int32)
    ec = -(-e // (NW * CHUNK))
    ec = -(-ec // 8) * 8
    e_pad = NW * ec * CHUNK
    src = jnp.pad(src, (0, e_pad - e)).reshape(NW * ec, CHUNK)
    dst = jnp.pad(dst, (0, e_pad - e), constant_values=n).reshape(NW * ec, CHUNK)

    Wn1p = jnp.zeros((d_in, D1P), jnp.float32).at[:, :100].set(Wn1)
    Ws1p = jnp.zeros((d_in, D1P), jnp.float32).at[:, :100].set(Ws1)
    b1p = jnp.zeros((1, D1P), jnp.float32).at[0, :100].set(b1)
    Wn2e = jnp.zeros((D1P, D2P), jnp.float32).at[:100, :20].set(Wn2).at[100, 20].set(1.0)
    Ws2p = jnp.zeros((D1P, D2P), jnp.float32).at[:100, :20].set(Ws2)
    b2p = jnp.zeros((1, D2P), jnp.float32).at[0, :20].set(b2)
    Wfc1p = jnp.zeros((D2P, 16), jnp.float32).at[:20, :10].set(W_fc1)
    bfc1p = jnp.zeros((1, 16), jnp.float32).at[0, :10].set(b_fc1)
    Wfc2p = jnp.zeros((16, 1), jnp.float32).at[:10, 0].set(W_fc2[:, 0])
    bfc2p = b_fc2.reshape(1, 1)

    bs_a = 512
    p1 = pl.pallas_call(
        _proj1_body,
        grid=(NP // bs_a,),
        in_specs=[
            pl.BlockSpec((bs_a, d_in), lambda i: (i, 0)),
            pl.BlockSpec((d_in, D1P), lambda i: (0, 0)),
        ],
        out_specs=pl.BlockSpec((bs_a, D1P), lambda i: (i, 0)),
        out_shape=jax.ShapeDtypeStruct((NP, D1P), jnp.float32),
    )(xp, Wn1p)

    agg1 = _sc_agg(p1, src, dst, D1P, ec)

    bs_b = 512
    p2, s2 = pl.pallas_call(
        _mid_body,
        grid=(NP // bs_b,),
        in_specs=[
            pl.BlockSpec((bs_b, d_in), lambda i: (i, 0)),
            pl.BlockSpec((NC, bs_b, D1P), lambda i: (0, i, 0)),
            pl.BlockSpec((d_in, D1P), lambda i: (0, 0)),
            pl.BlockSpec((1, D1P), lambda i: (0, 0)),
            pl.BlockSpec((D1P, D2P), lambda i: (0, 0)),
            pl.BlockSpec((D1P, D2P), lambda i: (0, 0)),
        ],
        out_specs=[
            pl.BlockSpec((bs_b, D2P), lambda i: (i, 0)),
            pl.BlockSpec((bs_b, D2P), lambda i: (i, 0)),
        ],
        out_shape=[
            jax.ShapeDtypeStruct((NP, D2P), jnp.float32),
            jax.ShapeDtypeStruct((NP, D2P), jnp.float32),
        ],
    )(xp, agg1, Ws1p, b1p, Wn2e, Ws2p)

    agg2 = _sc_agg(p2, src, dst, D2P, ec)

    bs_c = 1024
    ngrid = NP // bs_c
    _, out = pl.pallas_call(
        functools.partial(_tail_body, bs=bs_c, ngrid=ngrid),
        grid=(ngrid,),
        in_specs=[
            pl.BlockSpec((bs_c, D2P), lambda i: (i, 0)),
            pl.BlockSpec((NC, bs_c, D2P), lambda i: (0, i, 0)),
            pl.BlockSpec((1, D2P), lambda i: (0, 0)),
            pl.BlockSpec((D2P, 16), lambda i: (0, 0)),
            pl.BlockSpec((1, 16), lambda i: (0, 0)),
            pl.BlockSpec((16, 1), lambda i: (0, 0)),
            pl.BlockSpec((1, 1), lambda i: (0, 0)),
        ],
        out_specs=[
            pl.BlockSpec((1, D2P), lambda i: (0, 0)),
            pl.BlockSpec((1, 1), lambda i: (0, 0)),
        ],
        out_shape=[
            jax.ShapeDtypeStruct((1, D2P), jnp.float32),
            jax.ShapeDtypeStruct((1, 1), jnp.float32),
        ],
    )(s2, agg2, b2p, Wfc1p, bfc1p, Wfc2p, bfc2p)
    return out


# speedup vs baseline: 3.0993x; 1.0712x over previous
import functools

import jax
import jax.numpy as jnp
from jax import lax
from jax.experimental import pallas as pl
from jax.experimental.pallas import tpu as pltpu
from jax.experimental.pallas import tpu_sc as plsc

N_NODES = 10000
NP = 10240
D1P = 128
D2P = 128
CHUNK = 128
NC, NS = 2, 16
NW = NC * NS


def _sc_agg(p, src2d, dst2d, dp, ec):
    rows_per_tile = NP // NS
    n_zero = rows_per_tile // CHUNK
    G = 8
    NG = ec // G
    mesh = plsc.VectorSubcoreMesh(core_axis_name="c", subcore_axis_name="s")

    @functools.partial(
        pl.kernel,
        out_type=jax.ShapeDtypeStruct((NC, NP, dp), jnp.float32),
        mesh=mesh,
        scratch_types=[
            pltpu.VMEM((G, CHUNK), jnp.int32),
            pltpu.VMEM((G, CHUNK), jnp.int32),
            pltpu.VMEM((G, CHUNK), jnp.int32),
            pltpu.VMEM((G, CHUNK), jnp.int32),
            pltpu.VMEM((CHUNK, dp), jnp.float32),
            pltpu.VMEM((CHUNK, dp), jnp.float32),
            pltpu.VMEM_SHARED((NP, dp), jnp.float32),
            pltpu.SemaphoreType.DMA,
            pltpu.SemaphoreType.DMA,
            pltpu.SemaphoreType.DMA,
            pltpu.SemaphoreType.DMA,
        ],
    )
    def k(p_hbm, src_hbm, dst_hbm, out_hbm, srcA, dstA, srcB, dstB,
          rows_v, rows_w, acc, gsem0, gsem1, isem0, isem1):
        c = lax.axis_index("c")
        s = lax.axis_index("s")
        wid = s * NC + c

        zero16 = jnp.zeros((16,), jnp.float32)

        def zbody(r, carry):
            for jj in range(dp // 16):
                rows_v[r, pl.ds(jj * 16, 16)] = zero16
            return carry

        lax.fori_loop(0, CHUNK, zbody, 0)
        for b in range(n_zero):
            pltpu.sync_copy(rows_v, acc.at[pl.ds(s * rows_per_tile + b * CHUNK, CHUNK)])
        plsc.subcore_barrier()

        idx_bufs = [(srcA, dstA, isem0), (srcB, dstB, isem1)]

        def idx_load(g, sbuf, dbuf, isem):
            base = wid * ec + g * G
            return (pltpu.make_async_copy(src_hbm.at[pl.ds(base, G)], sbuf, isem),
                    pltpu.make_async_copy(dst_hbm.at[pl.ds(base, G)], dbuf, isem))

        def gather(sbuf, j, buf, sem):
            return pltpu.make_async_copy(p_hbm.at[sbuf.at[j]], buf, sem)

        for cp in idx_load(0, srcA, dstA, isem0):
            cp.start()

        for g in range(NG):
            sbuf, dbuf, isem = idx_bufs[g % 2]
            for cp in idx_load(g, sbuf, dbuf, isem):
                cp.wait()
            if g + 1 < NG:
                nsbuf, ndbuf, nisem = idx_bufs[(g + 1) % 2]
                for cp in idx_load(g + 1, nsbuf, ndbuf, nisem):
                    cp.start()

            gather(sbuf, 0, rows_v, gsem0).start()

            def pair(jj, carry, sbuf=sbuf, dbuf=dbuf):
                j0 = 2 * jj
                gather(sbuf, j0, rows_v, gsem0).wait()
                gather(sbuf, j0 + 1, rows_w, gsem1).start()
                pltpu.sync_copy(rows_v, acc.at[dbuf.at[j0]], add=True)
                gather(sbuf, j0 + 1, rows_w, gsem1).wait()
                gather(sbuf, j0 + 2, rows_v, gsem0).start()
                pltpu.sync_copy(rows_w, acc.at[dbuf.at[j0 + 1]], add=True)
                return carry

            lax.fori_loop(0, G // 2 - 1, pair, 0)
            jl = G - 2
            gather(sbuf, jl, rows_v, gsem0).wait()
            gather(sbuf, jl + 1, rows_w, gsem1).start()
            pltpu.sync_copy(rows_v, acc.at[dbuf.at[jl]], add=True)
            gather(sbuf, jl + 1, rows_w, gsem1).wait()
            pltpu.sync_copy(rows_w, acc.at[dbuf.at[jl + 1]], add=True)
        plsc.subcore_barrier()

        pltpu.sync_copy(
            acc.at[pl.ds(s * rows_per_tile, rows_per_tile)],
            out_hbm.at[c, pl.ds(s * rows_per_tile, rows_per_tile)],
        )

    return k(p, src2d, dst2d)


def _proj1_body(x_ref, w_ref, o_ref):
    ones_col = (lax.broadcasted_iota(jnp.int32, (1, D1P), 1) == 100).astype(jnp.float32)
    o_ref[...] = jnp.dot(x_ref[...], w_ref[...],
                         preferred_element_type=jnp.float32, precision=jax.lax.Precision.HIGHEST) + ones_col


def _mid_body(x_ref, agg_ref, ws1_ref, b1_ref, wn2_ref, ws2_ref, p2_ref, s2_ref):
    a = agg_ref[0] + agg_ref[1]
    deg = jnp.clip(a[:, 100:101], 1.0, None)
    mean1 = a / deg
    h1 = jnp.maximum(
        jnp.dot(x_ref[...], ws1_ref[...], preferred_element_type=jnp.float32, precision=jax.lax.Precision.HIGHEST)
        + b1_ref[...] + mean1, 0.0)
    is_ones = lax.broadcasted_iota(jnp.int32, h1.shape, 1) == 100
    h1 = jnp.where(is_ones, 1.0, h1)
    p2_ref[...] = jnp.dot(h1, wn2_ref[...], preferred_element_type=jnp.float32, precision=jax.lax.Precision.HIGHEST)
    s2_ref[...] = jnp.dot(h1, ws2_ref[...], preferred_element_type=jnp.float32, precision=jax.lax.Precision.HIGHEST)


def _tail_body(s2_ref, agg_ref, b2_ref, wfc1_ref, bfc1_ref, wfc2_ref, bfc2_ref,
               hg_ref, o_ref, *, bs, ngrid):
    i = pl.program_id(0)
    a = agg_ref[0] + agg_ref[1]
    deg = jnp.clip(a[:, 20:21], 1.0, None)
    h2 = jnp.maximum(s2_ref[...] + a / deg + b2_ref[...], 0.0)
    row = lax.broadcasted_iota(jnp.int32, h2.shape, 0) + i * bs
    h2 = jnp.where(row < N_NODES, h2, 0.0)
    psum = jnp.sum(h2, axis=0, keepdims=True)

    @pl.when(i == 0)
    def _():
        hg_ref[...] = psum

    @pl.when(i > 0)
    def _():
        hg_ref[...] = hg_ref[...] + psum

    @pl.when(i == ngrid - 1)
    def _():
        hg = hg_ref[...] * (1.0 / N_NODES)
        t = jnp.maximum(
            jnp.dot(hg, wfc1_ref[...], preferred_element_type=jnp.float32, precision=jax.lax.Precision.HIGHEST)
            + bfc1_ref[...], 0.0)
        o_ref[...] = (jnp.dot(t, wfc2_ref[...], preferred_element_type=jnp.float32, precision=jax.lax.Precision.HIGHEST)
                      + bfc2_ref[...])


def kernel(x, edge_index, Ws1, Wn1, b1, Ws2, Wn2, b2, W_fc1, b_fc1, W_fc2, b_fc2):
    n, d_in = x.shape
    e = edge_index.shape[1]

    xp = jnp.pad(x, ((0, NP - n), (0, 0)))
    src = edge_index[0].astype(jnp.int32)
    dst = edge_index[1].astype(jnp.int32)
    ec = -(-e // (NW * CHUNK))
    ec = -(-ec // 8) * 8
    e_pad = NW * ec * CHUNK
    src = jnp.pad(src, (0, e_pad - e)).reshape(NW * ec, CHUNK)
    dst = jnp.pad(dst, (0, e_pad - e), constant_values=n).reshape(NW * ec, CHUNK)

    Wn1p = jnp.zeros((d_in, D1P), jnp.float32).at[:, :100].set(Wn1)
    Ws1p = jnp.zeros((d_in, D1P), jnp.float32).at[:, :100].set(Ws1)
    b1p = jnp.zeros((1, D1P), jnp.float32).at[0, :100].set(b1)
    Wn2e = jnp.zeros((D1P, D2P), jnp.float32).at[:100, :20].set(Wn2).at[100, 20].set(1.0)
    Ws2p = jnp.zeros((D1P, D2P), jnp.float32).at[:100, :20].set(Ws2)
    b2p = jnp.zeros((1, D2P), jnp.float32).at[0, :20].set(b2)
    Wfc1p = jnp.zeros((D2P, 16), jnp.float32).at[:20, :10].set(W_fc1)
    bfc1p = jnp.zeros((1, 16), jnp.float32).at[0, :10].set(b_fc1)
    Wfc2p = jnp.zeros((16, 1), jnp.float32).at[:10, 0].set(W_fc2[:, 0])
    bfc2p = b_fc2.reshape(1, 1)

    bs_a = 512
    p1 = pl.pallas_call(
        _proj1_body,
        grid=(NP // bs_a,),
        in_specs=[
            pl.BlockSpec((bs_a, d_in), lambda i: (i, 0)),
            pl.BlockSpec((d_in, D1P), lambda i: (0, 0)),
        ],
        out_specs=pl.BlockSpec((bs_a, D1P), lambda i: (i, 0)),
        out_shape=jax.ShapeDtypeStruct((NP, D1P), jnp.float32),
    )(xp, Wn1p)

    agg1 = _sc_agg(p1, src, dst, D1P, ec)

    bs_b = 512
    p2, s2 = pl.pallas_call(
        _mid_body,
        grid=(NP // bs_b,),
        in_specs=[
            pl.BlockSpec((bs_b, d_in), lambda i: (i, 0)),
            pl.BlockSpec((NC, bs_b, D1P), lambda i: (0, i, 0)),
            pl.BlockSpec((d_in, D1P), lambda i: (0, 0)),
            pl.BlockSpec((1, D1P), lambda i: (0, 0)),
            pl.BlockSpec((D1P, D2P), lambda i: (0, 0)),
            pl.BlockSpec((D1P, D2P), lambda i: (0, 0)),
        ],
        out_specs=[
            pl.BlockSpec((bs_b, D2P), lambda i: (i, 0)),
            pl.BlockSpec((bs_b, D2P), lambda i: (i, 0)),
        ],
        out_shape=[
            jax.ShapeDtypeStruct((NP, D2P), jnp.float32),
            jax.ShapeDtypeStruct((NP, D2P), jnp.float32),
        ],
    )(xp, agg1, Ws1p, b1p, Wn2e, Ws2p)

    agg2 = _sc_agg(p2, src, dst, D2P, ec)

    bs_c = 1024
    ngrid = NP // bs_c
    _, out = pl.pallas_call(
        functools.partial(_tail_body, bs=bs_c, ngrid=ngrid),
        grid=(ngrid,),
        in_specs=[
            pl.BlockSpec((bs_c, D2P), lambda i: (i, 0)),
            pl.BlockSpec((NC, bs_c, D2P), lambda i: (0, i, 0)),
            pl.BlockSpec((1, D2P), lambda i: (0, 0)),
            pl.BlockSpec((D2P, 16), lambda i: (0, 0)),
            pl.BlockSpec((1, 16), lambda i: (0, 0)),
            pl.BlockSpec((16, 1), lambda i: (0, 0)),
            pl.BlockSpec((1, 1), lambda i: (0, 0)),
        ],
        out_specs=[
            pl.BlockSpec((1, D2P), lambda i: (0, 0)),
            pl.BlockSpec((1, 1), lambda i: (0, 0)),
        ],
        out_shape=[
            jax.ShapeDtypeStruct((1, D2P), jnp.float32),
            jax.ShapeDtypeStruct((1, 1), jnp.float32),
        ],
    )(s2, agg2, b2p, Wfc1p, bfc1p, Wfc2p, bfc2p)
    return out


# speedup vs baseline: 5.5119x; 1.7784x over previous
import functools

import jax
import jax.numpy as jnp
from jax import lax
from jax.experimental import pallas as pl
from jax.experimental.pallas import tpu as pltpu
from jax.experimental.pallas import tpu_sc as plsc

N_NODES = 10000
NP = 10240
D1P = 112
D2P = 32
CHUNK = 128
NC, NS = 2, 16
NW = NC * NS
NG0_FRAC = 0.5


def _sc_agg(p, src2d, dst2d, dp, ec, ng0_frac):
    rows_per_tile = NP // NS
    n_zero = rows_per_tile // CHUNK
    G = 8
    n_groups = NW * ec // (G * NS)
    NG0 = round(n_groups * ng0_frac)
    NG1 = n_groups - NG0
    mesh = plsc.VectorSubcoreMesh(core_axis_name="c", subcore_axis_name="s")

    @functools.partial(
        pl.kernel,
        out_type=jax.ShapeDtypeStruct((NC, NP, dp), jnp.float32),
        mesh=mesh,
        compiler_params=pltpu.CompilerParams(use_tc_tiling_on_sc=False),
        scratch_types=[
            pltpu.VMEM((G, CHUNK), jnp.int32),
            pltpu.VMEM((G, CHUNK), jnp.int32),
            pltpu.VMEM((G, CHUNK), jnp.int32),
            pltpu.VMEM((G, CHUNK), jnp.int32),
            pltpu.VMEM((CHUNK, dp), jnp.float32),
            pltpu.VMEM((CHUNK, dp), jnp.float32),
            pltpu.VMEM_SHARED((NP, dp), jnp.float32),
            pltpu.SemaphoreType.DMA,
            pltpu.SemaphoreType.DMA,
            pltpu.SemaphoreType.DMA,
            pltpu.SemaphoreType.DMA,
        ],
    )
    def k(p_hbm, src_hbm, dst_hbm, out_hbm, srcA, dstA, srcB, dstB,
          rows_v, rows_w, acc, gsem0, gsem1, isem0, isem1):
        c = lax.axis_index("c")
        s = lax.axis_index("s")
        wid = s * NC + c

        zero16 = jnp.zeros((16,), jnp.float32)

        def zbody(r, carry):
            for jj in range(dp // 16):
                rows_v[r, pl.ds(jj * 16, 16)] = zero16
            return carry

        lax.fori_loop(0, CHUNK, zbody, 0)
        for b in range(n_zero):
            pltpu.sync_copy(rows_v, acc.at[pl.ds(s * rows_per_tile + b * CHUNK, CHUNK)])
        plsc.subcore_barrier()

        idx_bufs = [(srcA, dstA, isem0), (srcB, dstB, isem1)]

        def gather(sbuf, j, buf, sem):
            return pltpu.make_async_copy(p_hbm.at[sbuf.at[j]], buf, sem)

        def run_edges(chunk_base, n_g):
            def idx_load(g, sbuf, dbuf, isem):
                base = chunk_base + g * G
                return (pltpu.make_async_copy(src_hbm.at[pl.ds(base, G)], sbuf, isem),
                        pltpu.make_async_copy(dst_hbm.at[pl.ds(base, G)], dbuf, isem))

            for cp in idx_load(0, srcA, dstA, isem0):
                cp.start()

            for g in range(n_g):
                sbuf, dbuf, isem = idx_bufs[g % 2]
                for cp in idx_load(g, sbuf, dbuf, isem):
                    cp.wait()
                if g + 1 < n_g:
                    nsbuf, ndbuf, nisem = idx_bufs[(g + 1) % 2]
                    for cp in idx_load(g + 1, nsbuf, ndbuf, nisem):
                        cp.start()

                gather(sbuf, 0, rows_v, gsem0).start()

                def pair(jj, carry, sbuf=sbuf, dbuf=dbuf):
                    j0 = 2 * jj
                    gather(sbuf, j0, rows_v, gsem0).wait()
                    gather(sbuf, j0 + 1, rows_w, gsem1).start()
                    pltpu.sync_copy(rows_v, acc.at[dbuf.at[j0]], add=True)
                    gather(sbuf, j0 + 1, rows_w, gsem1).wait()
                    gather(sbuf, j0 + 2, rows_v, gsem0).start()
                    pltpu.sync_copy(rows_w, acc.at[dbuf.at[j0 + 1]], add=True)
                    return carry

                lax.fori_loop(0, G // 2 - 1, pair, 0)
                jl = G - 2
                gather(sbuf, jl, rows_v, gsem0).wait()
                gather(sbuf, jl + 1, rows_w, gsem1).start()
                pltpu.sync_copy(rows_v, acc.at[dbuf.at[jl]], add=True)
                gather(sbuf, jl + 1, rows_w, gsem1).wait()
                pltpu.sync_copy(rows_w, acc.at[dbuf.at[jl + 1]], add=True)

        if NG0 > 0:
            @pl.when(c == 0)
            def _():
                run_edges(s * (NG0 * G), NG0)
        if NG1 > 0:
            @pl.when(c == 1)
            def _():
                run_edges(NS * NG0 * G + s * (NG1 * G), NG1)
        plsc.subcore_barrier()

        pltpu.sync_copy(
            acc.at[pl.ds(s * rows_per_tile, rows_per_tile)],
            out_hbm.at[c, pl.ds(s * rows_per_tile, rows_per_tile)],
        )

    return k(p, src2d, dst2d)


def _proj1_body(x_ref, w_ref, o_ref):
    ones_col = (lax.broadcasted_iota(jnp.int32, (1, D1P), 1) == 100).astype(jnp.float32)
    o_ref[...] = jnp.dot(x_ref[...], w_ref[...],
                         preferred_element_type=jnp.float32, precision=jax.lax.Precision.HIGHEST) + ones_col


def _mid_body(x_ref, agg_ref, ws1_ref, b1_ref, wn2_ref, ws2_ref, p2_ref, s2_ref):
    a = agg_ref[0] + agg_ref[1]
    deg = jnp.clip(a[:, 100:101], 1.0, None)
    mean1 = a / deg
    h1 = jnp.maximum(
        jnp.dot(x_ref[...], ws1_ref[...], preferred_element_type=jnp.float32, precision=jax.lax.Precision.HIGHEST)
        + b1_ref[...] + mean1, 0.0)
    is_ones = lax.broadcasted_iota(jnp.int32, h1.shape, 1) == 100
    h1 = jnp.where(is_ones, 1.0, h1)
    p2_ref[...] = jnp.dot(h1, wn2_ref[...], preferred_element_type=jnp.float32, precision=jax.lax.Precision.HIGHEST)
    s2_ref[...] = jnp.dot(h1, ws2_ref[...], preferred_element_type=jnp.float32, precision=jax.lax.Precision.HIGHEST)


def _tail_body(s2_ref, agg_ref, b2_ref, wfc1_ref, bfc1_ref, wfc2_ref, bfc2_ref,
               hg_ref, o_ref, *, bs, ngrid):
    i = pl.program_id(0)
    a = agg_ref[0] + agg_ref[1]
    deg = jnp.clip(a[:, 20:21], 1.0, None)
    h2 = jnp.maximum(s2_ref[...] + a / deg + b2_ref[...], 0.0)
    row = lax.broadcasted_iota(jnp.int32, h2.shape, 0) + i * bs
    h2 = jnp.where(row < N_NODES, h2, 0.0)
    psum = jnp.sum(h2, axis=0, keepdims=True)

    @pl.when(i == 0)
    def _():
        hg_ref[...] = psum

    @pl.when(i > 0)
    def _():
        hg_ref[...] = hg_ref[...] + psum

    @pl.when(i == ngrid - 1)
    def _():
        hg = hg_ref[...] * (1.0 / N_NODES)
        t = jnp.maximum(
            jnp.dot(hg, wfc1_ref[...], preferred_element_type=jnp.float32, precision=jax.lax.Precision.HIGHEST)
            + bfc1_ref[...], 0.0)
        o_ref[...] = (jnp.dot(t, wfc2_ref[...], preferred_element_type=jnp.float32, precision=jax.lax.Precision.HIGHEST)
                      + bfc2_ref[...])


def kernel(x, edge_index, Ws1, Wn1, b1, Ws2, Wn2, b2, W_fc1, b_fc1, W_fc2, b_fc2):
    n, d_in = x.shape
    e = edge_index.shape[1]

    xp = jnp.pad(x, ((0, NP - n), (0, 0)))
    src = edge_index[0].astype(jnp.int32)
    dst = edge_index[1].astype(jnp.int32)
    ec = -(-e // (NW * CHUNK))
    ec = -(-ec // 8) * 8
    e_pad = NW * ec * CHUNK
    src = jnp.pad(src, (0, e_pad - e)).reshape(NW * ec, CHUNK)
    dst = jnp.pad(dst, (0, e_pad - e), constant_values=n).reshape(NW * ec, CHUNK)

    Wn1p = jnp.zeros((d_in, D1P), jnp.float32).at[:, :100].set(Wn1)
    Ws1p = jnp.zeros((d_in, D1P), jnp.float32).at[:, :100].set(Ws1)
    b1p = jnp.zeros((1, D1P), jnp.float32).at[0, :100].set(b1)
    Wn2e = jnp.zeros((D1P, D2P), jnp.float32).at[:100, :20].set(Wn2).at[100, 20].set(1.0)
    Ws2p = jnp.zeros((D1P, D2P), jnp.float32).at[:100, :20].set(Ws2)
    b2p = jnp.zeros((1, D2P), jnp.float32).at[0, :20].set(b2)
    Wfc1p = jnp.zeros((D2P, 16), jnp.float32).at[:20, :10].set(W_fc1)
    bfc1p = jnp.zeros((1, 16), jnp.float32).at[0, :10].set(b_fc1)
    Wfc2p = jnp.zeros((16, 1), jnp.float32).at[:10, 0].set(W_fc2[:, 0])
    bfc2p = b_fc2.reshape(1, 1)

    bs_a = 512
    p1 = pl.pallas_call(
        _proj1_body,
        grid=(NP // bs_a,),
        in_specs=[
            pl.BlockSpec((bs_a, d_in), lambda i: (i, 0)),
            pl.BlockSpec((d_in, D1P), lambda i: (0, 0)),
        ],
        out_specs=pl.BlockSpec((bs_a, D1P), lambda i: (i, 0)),
        out_shape=jax.ShapeDtypeStruct((NP, D1P), jnp.float32),
    )(xp, Wn1p)

    agg1 = _sc_agg(p1, src, dst, D1P, ec, NG0_FRAC)

    bs_b = 512
    p2, s2 = pl.pallas_call(
        _mid_body,
        grid=(NP // bs_b,),
        in_specs=[
            pl.BlockSpec((bs_b, d_in), lambda i: (i, 0)),
            pl.BlockSpec((NC, bs_b, D1P), lambda i: (0, i, 0)),
            pl.BlockSpec((d_in, D1P), lambda i: (0, 0)),
            pl.BlockSpec((1, D1P), lambda i: (0, 0)),
            pl.BlockSpec((D1P, D2P), lambda i: (0, 0)),
            pl.BlockSpec((D1P, D2P), lambda i: (0, 0)),
        ],
        out_specs=[
            pl.BlockSpec((bs_b, D2P), lambda i: (i, 0)),
            pl.BlockSpec((bs_b, D2P), lambda i: (i, 0)),
        ],
        out_shape=[
            jax.ShapeDtypeStruct((NP, D2P), jnp.float32),
            jax.ShapeDtypeStruct((NP, D2P), jnp.float32),
        ],
    )(xp, agg1, Ws1p, b1p, Wn2e, Ws2p)

    agg2 = _sc_agg(p2, src, dst, D2P, ec, NG0_FRAC)

    bs_c = 1024
    ngrid = NP // bs_c
    _, out = pl.pallas_call(
        functools.partial(_tail_body, bs=bs_c, ngrid=ngrid),
        grid=(ngrid,),
        in_specs=[
            pl.BlockSpec((bs_c, D2P), lambda i: (i, 0)),
            pl.BlockSpec((NC, bs_c, D2P), lambda i: (0, i, 0)),
            pl.BlockSpec((1, D2P), lambda i: (0, 0)),
            pl.BlockSpec((D2P, 16), lambda i: (0, 0)),
            pl.BlockSpec((1, 16), lambda i: (0, 0)),
            pl.BlockSpec((16, 1), lambda i: (0, 0)),
            pl.BlockSpec((1, 1), lambda i: (0, 0)),
        ],
        out_specs=[
            pl.BlockSpec((1, D2P), lambda i: (0, 0)),
            pl.BlockSpec((1, 1), lambda i: (0, 0)),
        ],
        out_shape=[
            jax.ShapeDtypeStruct((1, D2P), jnp.float32),
            jax.ShapeDtypeStruct((1, 1), jnp.float32),
        ],
    )(s2, agg2, b2p, Wfc1p, bfc1p, Wfc2p, bfc2p)
    return out


# speedup vs baseline: 11.6692x; 2.1171x over previous
import functools

import jax
import jax.numpy as jnp
from jax import lax
from jax.experimental import pallas as pl
from jax.experimental.pallas import tpu as pltpu
from jax.experimental.pallas import tpu_sc as plsc

N_NODES = 10000
NP = 10240
D1P = 128
D2P = 32
CHUNK = 128
NC, NS = 2, 16
NW = NC * NS


def _sc_agg(p, src2d, dst2d, dp, ec):
    dph = dp // NC
    rows_per_tile = NP // NS
    n_zero = rows_per_tile // CHUNK
    G = 8
    n_chunks = NW * ec
    NG = n_chunks // (G * NS)
    mesh = plsc.VectorSubcoreMesh(core_axis_name="c", subcore_axis_name="s")

    @functools.partial(
        pl.kernel,
        out_type=jax.ShapeDtypeStruct((NC, NP, dph), jnp.float32),
        mesh=mesh,
        compiler_params=pltpu.CompilerParams(use_tc_tiling_on_sc=False),
        scratch_types=[
            pltpu.VMEM((G, CHUNK), jnp.int32),
            pltpu.VMEM((G, CHUNK), jnp.int32),
            pltpu.VMEM((G, CHUNK), jnp.int32),
            pltpu.VMEM((G, CHUNK), jnp.int32),
            pltpu.VMEM((CHUNK, dph), jnp.float32),
            pltpu.VMEM((CHUNK, dph), jnp.float32),
            pltpu.VMEM_SHARED((NP, dph), jnp.float32),
            pltpu.VMEM_SHARED((NP, dph), jnp.float32),
            pltpu.SemaphoreType.DMA,
            pltpu.SemaphoreType.DMA,
            pltpu.SemaphoreType.DMA,
            pltpu.SemaphoreType.DMA,
        ],
    )
    def k(p_hbm, src_hbm, dst_hbm, out_hbm, srcA, dstA, srcB, dstB,
          rows_v, rows_w, stage, acc, gsem0, gsem1, isem0, isem1):
        c = lax.axis_index("c")
        s = lax.axis_index("s")

        r0 = s * rows_per_tile
        pltpu.sync_copy(
            p_hbm.at[pl.ds(r0, rows_per_tile), pl.ds(c * dph, dph)],
            stage.at[pl.ds(r0, rows_per_tile)],
        )

        zero16 = jnp.zeros((16,), jnp.float32)

        def zbody(r, carry):
            for jj in range(dph // 16):
                rows_v[r, pl.ds(jj * 16, 16)] = zero16
            return carry

        lax.fori_loop(0, CHUNK, zbody, 0)
        for b in range(n_zero):
            pltpu.sync_copy(rows_v, acc.at[pl.ds(r0 + b * CHUNK, CHUNK)])
        plsc.subcore_barrier()

        idx_bufs = [(srcA, dstA, isem0), (srcB, dstB, isem1)]

        def gather(sbuf, j, buf, sem):
            return pltpu.make_async_copy(stage.at[sbuf.at[j]], buf, sem)

        chunk_base = s * (NG * G)

        def idx_load(g, sbuf, dbuf, isem):
            base = chunk_base + g * G
            return (pltpu.make_async_copy(src_hbm.at[pl.ds(base, G)], sbuf, isem),
                    pltpu.make_async_copy(dst_hbm.at[pl.ds(base, G)], dbuf, isem))

        for cp in idx_load(0, srcA, dstA, isem0):
            cp.start()

        for g in range(NG):
            sbuf, dbuf, isem = idx_bufs[g % 2]
            for cp in idx_load(g, sbuf, dbuf, isem):
                cp.wait()
            if g + 1 < NG:
                nsbuf, ndbuf, nisem = idx_bufs[(g + 1) % 2]
                for cp in idx_load(g + 1, nsbuf, ndbuf, nisem):
                    cp.start()

            gather(sbuf, 0, rows_v, gsem0).start()

            def pair(jj, carry, sbuf=sbuf, dbuf=dbuf):
                j0 = 2 * jj
                gather(sbuf, j0, rows_v, gsem0).wait()
                gather(sbuf, j0 + 1, rows_w, gsem1).start()
                pltpu.sync_copy(rows_v, acc.at[dbuf.at[j0]], add=True)
                gather(sbuf, j0 + 1, rows_w, gsem1).wait()
                gather(sbuf, j0 + 2, rows_v, gsem0).start()
                pltpu.sync_copy(rows_w, acc.at[dbuf.at[j0 + 1]], add=True)
                return carry

            lax.fori_loop(0, G // 2 - 1, pair, 0)
            jl = G - 2
            gather(sbuf, jl, rows_v, gsem0).wait()
            gather(sbuf, jl + 1, rows_w, gsem1).start()
            pltpu.sync_copy(rows_v, acc.at[dbuf.at[jl]], add=True)
            gather(sbuf, jl + 1, rows_w, gsem1).wait()
            pltpu.sync_copy(rows_w, acc.at[dbuf.at[jl + 1]], add=True)
        plsc.subcore_barrier()

        pltpu.sync_copy(
            acc.at[pl.ds(r0, rows_per_tile)],
            out_hbm.at[c, pl.ds(r0, rows_per_tile)],
        )

    return k(p, src2d, dst2d)


def _proj1_body(x_ref, w_ref, o_ref):
    ones_col = (lax.broadcasted_iota(jnp.int32, (1, D1P), 1) == 100).astype(jnp.float32)
    o_ref[...] = jnp.dot(x_ref[...], w_ref[...],
                         preferred_element_type=jnp.float32, precision=jax.lax.Precision.HIGHEST) + ones_col


def _mid_body(x_ref, agg_ref, ws1_ref, b1_ref, wn2_ref, ws2_ref, p2_ref, s2_ref):
    a = jnp.concatenate([agg_ref[0], agg_ref[1]], axis=1)
    deg = jnp.clip(a[:, 100:101], 1.0, None)
    mean1 = a / deg
    h1 = jnp.maximum(
        jnp.dot(x_ref[...], ws1_ref[...], preferred_element_type=jnp.float32, precision=jax.lax.Precision.HIGHEST)
        + b1_ref[...] + mean1, 0.0)
    is_ones = lax.broadcasted_iota(jnp.int32, h1.shape, 1) == 100
    h1 = jnp.where(is_ones, 1.0, h1)
    p2_ref[...] = jnp.dot(h1, wn2_ref[...], preferred_element_type=jnp.float32, precision=jax.lax.Precision.HIGHEST)
    s2_ref[...] = jnp.dot(h1, ws2_ref[...], preferred_element_type=jnp.float32, precision=jax.lax.Precision.HIGHEST)


def _tail_body(s2_ref, agg_ref, b2_ref, wfc1_ref, bfc1_ref, wfc2_ref, bfc2_ref,
               hg_ref, o_ref, *, bs, ngrid):
    i = pl.program_id(0)
    a = jnp.concatenate([agg_ref[0], agg_ref[1]], axis=1)
    deg = jnp.clip(a[:, 20:21], 1.0, None)
    h2 = jnp.maximum(s2_ref[...] + a / deg + b2_ref[...], 0.0)
    row = lax.broadcasted_iota(jnp.int32, h2.shape, 0) + i * bs
    h2 = jnp.where(row < N_NODES, h2, 0.0)
    psum = jnp.sum(h2, axis=0, keepdims=True)

    @pl.when(i == 0)
    def _():
        hg_ref[...] = psum

    @pl.when(i > 0)
    def _():
        hg_ref[...] = hg_ref[...] + psum

    @pl.when(i == ngrid - 1)
    def _():
        hg = hg_ref[...] * (1.0 / N_NODES)
        t = jnp.maximum(
            jnp.dot(hg, wfc1_ref[...], preferred_element_type=jnp.float32, precision=jax.lax.Precision.HIGHEST)
            + bfc1_ref[...], 0.0)
        o_ref[...] = (jnp.dot(t, wfc2_ref[...], preferred_element_type=jnp.float32, precision=jax.lax.Precision.HIGHEST)
                      + bfc2_ref[...])


def kernel(x, edge_index, Ws1, Wn1, b1, Ws2, Wn2, b2, W_fc1, b_fc1, W_fc2, b_fc2):
    n, d_in = x.shape
    e = edge_index.shape[1]

    xp = jnp.pad(x, ((0, NP - n), (0, 0)))
    src = edge_index[0].astype(jnp.int32)
    dst = edge_index[1].astype(jnp.int32)
    ec = -(-e // (NW * CHUNK))
    ec = -(-ec // 8) * 8
    e_pad = NW * ec * CHUNK
    src = jnp.pad(src, (0, e_pad - e)).reshape(NW * ec, CHUNK)
    dst = jnp.pad(dst, (0, e_pad - e), constant_values=n).reshape(NW * ec, CHUNK)

    Wn1p = jnp.zeros((d_in, D1P), jnp.float32).at[:, :100].set(Wn1)
    Ws1p = jnp.zeros((d_in, D1P), jnp.float32).at[:, :100].set(Ws1)
    b1p = jnp.zeros((1, D1P), jnp.float32).at[0, :100].set(b1)
    Wn2e = jnp.zeros((D1P, D2P), jnp.float32).at[:100, :20].set(Wn2).at[100, 20].set(1.0)
    Ws2p = jnp.zeros((D1P, D2P), jnp.float32).at[:100, :20].set(Ws2)
    b2p = jnp.zeros((1, D2P), jnp.float32).at[0, :20].set(b2)
    Wfc1p = jnp.zeros((D2P, 16), jnp.float32).at[:20, :10].set(W_fc1)
    bfc1p = jnp.zeros((1, 16), jnp.float32).at[0, :10].set(b_fc1)
    Wfc2p = jnp.zeros((16, 1), jnp.float32).at[:10, 0].set(W_fc2[:, 0])
    bfc2p = b_fc2.reshape(1, 1)

    bs_a = 512
    p1 = pl.pallas_call(
        _proj1_body,
        grid=(NP // bs_a,),
        in_specs=[
            pl.BlockSpec((bs_a, d_in), lambda i: (i, 0)),
            pl.BlockSpec((d_in, D1P), lambda i: (0, 0)),
        ],
        out_specs=pl.BlockSpec((bs_a, D1P), lambda i: (i, 0)),
        out_shape=jax.ShapeDtypeStruct((NP, D1P), jnp.float32),
    )(xp, Wn1p)

    agg1 = _sc_agg(p1, src, dst, D1P, ec)

    bs_b = 512
    p2, s2 = pl.pallas_call(
        _mid_body,
        grid=(NP // bs_b,),
        in_specs=[
            pl.BlockSpec((bs_b, d_in), lambda i: (i, 0)),
            pl.BlockSpec((NC, bs_b, D1P // 2), lambda i: (0, i, 0)),
            pl.BlockSpec((d_in, D1P), lambda i: (0, 0)),
            pl.BlockSpec((1, D1P), lambda i: (0, 0)),
            pl.BlockSpec((D1P, D2P), lambda i: (0, 0)),
            pl.BlockSpec((D1P, D2P), lambda i: (0, 0)),
        ],
        out_specs=[
            pl.BlockSpec((bs_b, D2P), lambda i: (i, 0)),
            pl.BlockSpec((bs_b, D2P), lambda i: (i, 0)),
        ],
        out_shape=[
            jax.ShapeDtypeStruct((NP, D2P), jnp.float32),
            jax.ShapeDtypeStruct((NP, D2P), jnp.float32),
        ],
    )(xp, agg1, Ws1p, b1p, Wn2e, Ws2p)

    agg2 = _sc_agg(p2, src, dst, D2P, ec)

    bs_c = 1024
    ngrid = NP // bs_c
    _, out = pl.pallas_call(
        functools.partial(_tail_body, bs=bs_c, ngrid=ngrid),
        grid=(ngrid,),
        in_specs=[
            pl.BlockSpec((bs_c, D2P), lambda i: (i, 0)),
            pl.BlockSpec((NC, bs_c, D2P // 2), lambda i: (0, i, 0)),
            pl.BlockSpec((1, D2P), lambda i: (0, 0)),
            pl.BlockSpec((D2P, 16), lambda i: (0, 0)),
            pl.BlockSpec((1, 16), lambda i: (0, 0)),
            pl.BlockSpec((16, 1), lambda i: (0, 0)),
            pl.BlockSpec((1, 1), lambda i: (0, 0)),
        ],
        out_specs=[
            pl.BlockSpec((1, D2P), lambda i: (0, 0)),
            pl.BlockSpec((1, 1), lambda i: (0, 0)),
        ],
        out_shape=[
            jax.ShapeDtypeStruct((1, D2P), jnp.float32),
            jax.ShapeDtypeStruct((1, 1), jnp.float32),
        ],
    )(s2, agg2, b2p, Wfc1p, bfc1p, Wfc2p, bfc2p)
    return out
